# Initial kernel scaffold; baseline (speedup 1.0000x reference)
import jax
import jax.numpy as jnp
from jax.experimental import pallas as pl


def kernel(rho, xs):
    raise NotImplementedError("write your pallas kernel here")


# speedup vs baseline: 5.9184x; 5.9184x over previous
import functools

import jax
import jax.numpy as jnp
from jax import lax
from jax.experimental import pallas as pl
from jax.experimental.pallas import tpu as pltpu
from jax.experimental.pallas import tpu_sc as plsc

N1 = N2 = N3 = 256
SLICE_W = N2 * N3
NC, NS, L = 2, 16, 16
NW = NC * NS
CS = 26
NQ = -(-N1 // CS)
NP = -(-NQ // NC)
CHUNK = CS * SLICE_W
B = 1024
G = B // L


def _floor16(pos):
    t = pos.astype(jnp.int32)
    tf = t.astype(jnp.float32)
    gt = tf > pos
    bi = jnp.where(gt, t - 1, t)
    bf = jnp.where(gt, tf - 1.0, tf)
    return bi, pos - bf


def _make_grid_kernel(MP):
    NB = MP // (NS * B)
    PPW = NB * B
    mesh = plsc.VectorSubcoreMesh(
        core_axis_name="c", subcore_axis_name="s",
        num_cores=NC, num_subcores=NS)

    @functools.partial(
        pl.kernel,
        out_type=jax.ShapeDtypeStruct((N1 * N2 * N3,), jnp.float32),
        mesh=mesh,
        scratch_types=[
            pltpu.VMEM_SHARED((CHUNK,), jnp.float32),
            pltpu.VMEM((B,), jnp.float32),
            pltpu.VMEM((B,), jnp.float32),
            pltpu.VMEM((B,), jnp.float32),
            pltpu.VMEM((B,), jnp.float32),
            pltpu.VMEM((G, 8 * L), jnp.int32),
            pltpu.VMEM((G, 8 * L), jnp.float32),
            pltpu.VMEM((2048,), jnp.float32),
            pltpu.SemaphoreType.DMA,
        ],
    )
    def grid_kernel(xh, yh, zh, rho_p, out, chunk, xv, yv, zv, rv, idx_s,
                    w_s, zbuf, sem):
        c = lax.axis_index("c")
        s = lax.axis_index("s")
        iota = lax.iota(jnp.int32, L)
        zeros16 = jnp.zeros((L,), jnp.float32)

        def zb_body(i, carry):
            zbuf[pl.ds(i * L, L)] = zeros16
            return carry
        lax.fori_loop(0, 2048 // L, zb_body, 0)

        def pass_body(p, carry):
            q = p * NC + c
            cb = q * CHUNK
            qcs = q * CS
            limx = jnp.minimum(CS, N1 - qcs)

            def zc_body(i, carry2):
                pltpu.sync_copy(
                    zbuf,
                    chunk.at[pl.ds(s * (CHUNK // NS) + i * 2048, 2048)])
                return carry2
            lax.fori_loop(0, CHUNK // NS // 2048, zc_body, 0)
            plsc.subcore_barrier()

            fb_base = s * (CHUNK // NS)

            def block_body(b, carry2):
                base = s * PPW + b * B
                pltpu.sync_copy(xh.at[pl.ds(base, B)], xv)
                pltpu.sync_copy(yh.at[pl.ds(base, B)], yv)
                pltpu.sync_copy(zh.at[pl.ds(base, B)], zv)
                pltpu.sync_copy(rho_p.at[pl.ds(base, B)], rv)

                def group_body(j, carry3):
                    sl = pl.ds(j * L, L)
                    x = xv[sl]
                    y = yv[sl]
                    z = zv[sl]
                    r = rv[sl]
                    bix, dx = _floor16(x * 256.0 - 0.5)
                    biy, dy = _floor16(y * 256.0 - 0.5)
                    biz, dz = _floor16(z * 256.0 - 0.5)
                    wx0 = 1.0 - dx
                    ax0 = r * wx0
                    ax1 = r * dx
                    wy0 = 1.0 - dy
                    wz0 = 1.0 - dz
                    t0 = bix - qcs
                    t1 = t0 + 1
                    vx0 = (t0 >= 0) & (t0 < limx)
                    vx1 = (t1 >= 0) & (t1 < limx)
                    vy0 = (biy >= 0) & (biy < N2)
                    vy1 = (biy >= -1) & (biy < N2 - 1)
                    vz0 = (biz >= 0) & (biz < N3)
                    vz1 = (biz >= -1) & (biz < N3 - 1)
                    gx0 = (bix << 16) - cb
                    gx1 = gx0 + SLICE_W
                    gy0 = biy << 8
                    gy1 = gy0 + N3
                    fb = fb_base + j * L + iota
                    for k, (vx, vy, vz, gx, gy, axw, wy) in enumerate((
                            (vx0, vy0, vz0, gx0, gy0, ax0, wy0),
                            (vx0, vy0, vz1, gx0, gy0, ax0, wy0),
                            (vx0, vy1, vz0, gx0, gy1, ax0, dy),
                            (vx0, vy1, vz1, gx0, gy1, ax0, dy),
                            (vx1, vy0, vz0, gx1, gy0, ax1, wy0),
                            (vx1, vy0, vz1, gx1, gy0, ax1, wy0),
                            (vx1, vy1, vz0, gx1, gy1, ax1, dy),
                            (vx1, vy1, vz1, gx1, gy1, ax1, dy),
                    )):
                        cz = k & 1
                        v = vx & vy & vz
                        w = (axw * wy) * (dz if cz else wz0)
                        g = gx + gy + (biz + cz)
                        idx_s[j, pl.ds(k * L, L)] = jnp.where(v, g, fb)
                        w_s[j, pl.ds(k * L, L)] = jnp.where(v, w, 0.0)
                    pltpu.async_copy(
                        w_s.at[j], chunk.at[idx_s.at[j]], sem,
                        add=True).wait()
                    return carry3

                lax.fori_loop(0, G, group_body, 0)
                return carry2

            lax.fori_loop(0, NB, block_body, 0)
            plsc.subcore_barrier()

            def out_body(i, carry2):
                gs = qcs + i

                @pl.when(gs < N1)
                def _copy():
                    tw = SLICE_W // NS
                    pltpu.sync_copy(
                        chunk.at[pl.ds(i * SLICE_W + s * tw, tw)],
                        out.at[pl.ds(gs * SLICE_W + s * tw, tw)])
                return carry2

            lax.fori_loop(0, CS, out_body, 0)
            plsc.subcore_barrier()
            return carry

        lax.fori_loop(0, NP, pass_body, 0)

    return grid_kernel


def kernel(rho, xs):
    m = rho.shape[0]
    nb = -(-m // (NS * B))
    mp = nb * B * NS
    pad = mp - m
    rho_p = jnp.concatenate([rho, jnp.zeros((pad,), jnp.float32)])
    xs_p = jnp.concatenate([xs, jnp.zeros((pad, 3), jnp.float32)])
    xst = xs_p.T
    vol = _make_grid_kernel(mp)(xst[0], xst[1], xst[2], rho_p)
    return vol.reshape(N1, N2, N3)


# speedup vs baseline: 13.6383x; 2.3044x over previous
import functools

import jax
import jax.numpy as jnp
from jax import lax
from jax.experimental import pallas as pl
from jax.experimental.pallas import tpu as pltpu
from jax.experimental.pallas import tpu_sc as plsc

N1 = N2 = N3 = 256
SLICE_W = N2 * N3
NC, NS, L = 2, 16, 16
NW = NC * NS
CS = 26
NQ = -(-N1 // CS)
NP = -(-NQ // NC)
CHUNK = CS * SLICE_W
B = 1024
G = B // L


def _floor16(pos):
    t = pos.astype(jnp.int32)
    tf = t.astype(jnp.float32)
    gt = tf > pos
    bi = jnp.where(gt, t - 1, t)
    bf = jnp.where(gt, tf - 1.0, tf)
    return bi, pos - bf


def _make_grid_kernel(MP):
    NB = MP // (NS * B)
    PPW = NB * B
    mesh = plsc.VectorSubcoreMesh(
        core_axis_name="c", subcore_axis_name="s",
        num_cores=NC, num_subcores=NS)

    @functools.partial(
        pl.kernel,
        out_type=jax.ShapeDtypeStruct((N1 * N2 * N3,), jnp.float32),
        mesh=mesh,
        scratch_types=[
            pltpu.VMEM_SHARED((CHUNK,), jnp.float32),
            pltpu.VMEM((B,), jnp.float32),
            pltpu.VMEM((B,), jnp.float32),
            pltpu.VMEM((B,), jnp.float32),
            pltpu.VMEM((B,), jnp.float32),
            pltpu.VMEM((G, 8 * L), jnp.int32),
            pltpu.VMEM((G, 8 * L), jnp.float32),
            pltpu.VMEM((2048,), jnp.float32),
            pltpu.SemaphoreType.DMA,
            pltpu.SemaphoreType.DMA,
        ],
    )
    def grid_kernel(xh, yh, zh, rho_p, out, chunk, xv, yv, zv, rv, idx_s,
                    w_s, zbuf, sem, lsem):
        c = lax.axis_index("c")
        s = lax.axis_index("s")
        iota = lax.iota(jnp.int32, L)
        zeros16 = jnp.zeros((L,), jnp.float32)

        def zb_body(i, carry):
            zbuf[pl.ds(i * L, L)] = zeros16
            return carry
        lax.fori_loop(0, 2048 // L, zb_body, 0)

        def pass_body(p, carry):
            q = p * NC + c
            cb = q * CHUNK
            qcs = q * CS
            limx = jnp.minimum(CS, N1 - qcs)

            def zc_body(i, carry2):
                pltpu.sync_copy(
                    zbuf,
                    chunk.at[pl.ds(s * (CHUNK // NS) + i * 2048, 2048)])
                return carry2
            lax.fori_loop(0, CHUNK // NS // 2048, zc_body, 0)
            plsc.subcore_barrier()

            fb_base = s * (CHUNK // NS)

            def block_body(b, carry2):
                base = s * PPW + b * B
                bsl = pl.ds(base, B)
                d1 = pltpu.async_copy(xh.at[bsl], xv, lsem)
                d2 = pltpu.async_copy(yh.at[bsl], yv, lsem)
                d3 = pltpu.async_copy(zh.at[bsl], zv, lsem)
                d4 = pltpu.async_copy(rho_p.at[bsl], rv, lsem)
                d1.wait()
                d2.wait()
                d3.wait()
                d4.wait()

                def group_body(j, carry3):
                    sl = pl.ds(j * L, L)
                    x = xv[sl]
                    y = yv[sl]
                    z = zv[sl]
                    r = rv[sl]
                    bix, dx = _floor16(x * 256.0 - 0.5)
                    biy, dy = _floor16(y * 256.0 - 0.5)
                    biz, dz = _floor16(z * 256.0 - 0.5)
                    wx0 = 1.0 - dx
                    ax0 = r * wx0
                    ax1 = r * dx
                    wy0 = 1.0 - dy
                    wz0 = 1.0 - dz
                    t0 = bix - qcs
                    t1 = t0 + 1
                    vx0 = (t0 >= 0) & (t0 < limx)
                    vx1 = (t1 >= 0) & (t1 < limx)
                    vy0 = (biy >= 0) & (biy < N2)
                    vy1 = (biy >= -1) & (biy < N2 - 1)
                    vz0 = (biz >= 0) & (biz < N3)
                    vz1 = (biz >= -1) & (biz < N3 - 1)
                    gx0 = (bix << 16) - cb
                    gx1 = gx0 + SLICE_W
                    gy0 = biy << 8
                    gy1 = gy0 + N3
                    fb = fb_base + j * L + iota
                    for k, (vx, vy, vz, gx, gy, axw, wy) in enumerate((
                            (vx0, vy0, vz0, gx0, gy0, ax0, wy0),
                            (vx0, vy0, vz1, gx0, gy0, ax0, wy0),
                            (vx0, vy1, vz0, gx0, gy1, ax0, dy),
                            (vx0, vy1, vz1, gx0, gy1, ax0, dy),
                            (vx1, vy0, vz0, gx1, gy0, ax1, wy0),
                            (vx1, vy0, vz1, gx1, gy0, ax1, wy0),
                            (vx1, vy1, vz0, gx1, gy1, ax1, dy),
                            (vx1, vy1, vz1, gx1, gy1, ax1, dy),
                    )):
                        cz = k & 1
                        v = vx & vy & vz
                        w = (axw * wy) * (dz if cz else wz0)
                        g = gx + gy + (biz + cz)
                        idx_s[j, pl.ds(k * L, L)] = jnp.where(v, g, fb)
                        w_s[j, pl.ds(k * L, L)] = jnp.where(v, w, 0.0)
                    pltpu.async_copy(
                        w_s.at[j], chunk.at[idx_s.at[j]], sem, add=True)

                    @pl.when(j >= 3)
                    def _wait_lag():
                        pltpu.make_async_copy(
                            w_s.at[j - 3], chunk.at[idx_s.at[j - 3]],
                            sem).wait()
                    return carry3

                lax.fori_loop(0, G, group_body, 0)
                for jt in range(G - 3, G):
                    pltpu.make_async_copy(
                        w_s.at[jt], chunk.at[idx_s.at[jt]], sem).wait()
                return carry2

            lax.fori_loop(0, NB, block_body, 0)
            plsc.subcore_barrier()

            def out_body(i, carry2):
                gs = qcs + i

                @pl.when(gs < N1)
                def _copy():
                    tw = SLICE_W // NS
                    pltpu.sync_copy(
                        chunk.at[pl.ds(i * SLICE_W + s * tw, tw)],
                        out.at[pl.ds(gs * SLICE_W + s * tw, tw)])
                return carry2

            lax.fori_loop(0, CS, out_body, 0)
            plsc.subcore_barrier()
            return carry

        lax.fori_loop(0, NP, pass_body, 0)

    return grid_kernel


def kernel(rho, xs):
    m = rho.shape[0]
    nb = -(-m // (NS * B))
    mp = nb * B * NS
    pad = mp - m
    rho_p = jnp.concatenate([rho, jnp.zeros((pad,), jnp.float32)])
    xs_p = jnp.concatenate([xs, jnp.zeros((pad, 3), jnp.float32)])
    xst = xs_p.T
    vol = _make_grid_kernel(mp)(xst[0], xst[1], xst[2], rho_p)
    return vol.reshape(N1, N2, N3)


# speedup vs baseline: 22.5748x; 1.6553x over previous
import functools

import jax
import jax.numpy as jnp
from jax import lax
from jax.experimental import pallas as pl
from jax.experimental.pallas import tpu as pltpu
from jax.experimental.pallas import tpu_sc as plsc

N1 = N2 = N3 = 256
SLICE_W = N2 * N3
NC, NS, L = 2, 16, 16
NW = NC * NS
CS = 26
NQ = -(-N1 // CS)
NP = -(-NQ // NC)
CHUNK = CS * SLICE_W
B = 1024
G = B // L


def _floor16(pos):
    t = pos.astype(jnp.int32)
    tf = t.astype(jnp.float32)
    gt = tf > pos
    bi = jnp.where(gt, t - 1, t)
    bf = jnp.where(gt, tf - 1.0, tf)
    return bi, pos - bf


def _make_grid_kernel(MP):
    NB = MP // (NS * B)
    PPW = NB * B
    mesh = plsc.VectorSubcoreMesh(
        core_axis_name="c", subcore_axis_name="s",
        num_cores=NC, num_subcores=NS)

    @functools.partial(
        pl.kernel,
        out_type=jax.ShapeDtypeStruct((N1 * N2 * N3,), jnp.float32),
        mesh=mesh,
        scratch_types=[
            pltpu.VMEM_SHARED((CHUNK,), jnp.float32),
            pltpu.VMEM((B,), jnp.float32),
            pltpu.VMEM((B,), jnp.float32),
            pltpu.VMEM((B,), jnp.float32),
            pltpu.VMEM((B,), jnp.float32),
            pltpu.VMEM((G, 8 * L), jnp.int32),
            pltpu.VMEM((G, 8 * L), jnp.float32),
            pltpu.VMEM((2048,), jnp.float32),
            pltpu.VMEM((B,), jnp.int32),
            pltpu.SemaphoreType.DMA,
            pltpu.SemaphoreType.DMA,
        ],
        compiler_params=pltpu.CompilerParams(needs_layout_passes=False),
    )
    def grid_kernel(xh, yh, zh, rho_p, out, chunk, xv, yv, zv, rv, idx_s,
                    w_s, zbuf, cidx, sem, lsem):
        c = lax.axis_index("c")
        s = lax.axis_index("s")
        iota = lax.iota(jnp.int32, L)
        zeros16 = jnp.zeros((L,), jnp.float32)

        def zb_body(i, carry):
            zbuf[pl.ds(i * L, L)] = zeros16
            return carry
        lax.fori_loop(0, 2048 // L, zb_body, 0)

        def pass_body(p, carry):
            q = p * NC + c
            cb = q * CHUNK
            qcs = q * CS
            limx = jnp.minimum(CS, N1 - qcs)

            def zc_body(i, carry2):
                pltpu.sync_copy(
                    zbuf,
                    chunk.at[pl.ds(s * (CHUNK // NS) + i * 2048, 2048)])
                return carry2
            lax.fori_loop(0, CHUNK // NS // 2048, zc_body, 0)
            plsc.subcore_barrier()

            fb_base = s * (CHUNK // NS)

            def block_body(b, carry2):
                base = s * PPW + b * B
                bsl = pl.ds(base, B)
                d1 = pltpu.async_copy(xh.at[bsl], xv, lsem)
                d2 = pltpu.async_copy(yh.at[bsl], yv, lsem)
                d3 = pltpu.async_copy(zh.at[bsl], zv, lsem)
                d4 = pltpu.async_copy(rho_p.at[bsl], rv, lsem)
                d1.wait()
                d2.wait()
                d3.wait()
                d4.wait()

                def scan_body(j, cnt):
                    x = xv[pl.ds(j * L, L)]
                    pos = x * 256.0 - 0.5
                    t = pos.astype(jnp.int32)
                    tf = t.astype(jnp.float32)
                    bix = jnp.where(tf > pos, t - 1, t)
                    t0 = bix - qcs
                    rel = (t0 >= -1) & (t0 < limx)
                    pfx = plsc.cumsum(rel.astype(jnp.int32))
                    plsc.store_scatter(
                        cidx, [cnt + pfx - 1], j * L + iota, mask=rel)
                    return cnt + jnp.max(pfx)

                cnt = lax.fori_loop(0, G, scan_body, jnp.int32(0))
                ng = (cnt + (L - 1)) // L

                def group_body(j, carry3):
                    inb = (j * L + iota) < cnt
                    ids = jnp.where(inb, cidx[pl.ds(j * L, L)], 0)
                    x = plsc.load_gather(xv, [ids])
                    y = plsc.load_gather(yv, [ids])
                    z = plsc.load_gather(zv, [ids])
                    r = plsc.load_gather(rv, [ids])
                    bix, dx = _floor16(x * 256.0 - 0.5)
                    biy, dy = _floor16(y * 256.0 - 0.5)
                    biz, dz = _floor16(z * 256.0 - 0.5)
                    wx0 = 1.0 - dx
                    ax0 = r * wx0
                    ax1 = r * dx
                    wy0 = 1.0 - dy
                    wz0 = 1.0 - dz
                    t0 = bix - qcs
                    t1 = t0 + 1
                    vx0 = (t0 >= 0) & (t0 < limx)
                    vx1 = (t1 >= 0) & (t1 < limx)
                    vy0 = (biy >= 0) & (biy < N2)
                    vy1 = (biy >= -1) & (biy < N2 - 1)
                    vz0 = (biz >= 0) & (biz < N3)
                    vz1 = (biz >= -1) & (biz < N3 - 1)
                    gx0 = (bix << 16) - cb
                    gx1 = gx0 + SLICE_W
                    gy0 = biy << 8
                    gy1 = gy0 + N3
                    fb = fb_base + j * L + iota
                    vmask = inb
                    for k, (vx, vy, vz, gx, gy, axw, wy) in enumerate((
                            (vx0, vy0, vz0, gx0, gy0, ax0, wy0),
                            (vx0, vy0, vz1, gx0, gy0, ax0, wy0),
                            (vx0, vy1, vz0, gx0, gy1, ax0, dy),
                            (vx0, vy1, vz1, gx0, gy1, ax0, dy),
                            (vx1, vy0, vz0, gx1, gy0, ax1, wy0),
                            (vx1, vy0, vz1, gx1, gy0, ax1, wy0),
                            (vx1, vy1, vz0, gx1, gy1, ax1, dy),
                            (vx1, vy1, vz1, gx1, gy1, ax1, dy),
                    )):
                        cz = k & 1
                        v = vx & vy & vz & vmask
                        w = (axw * wy) * (dz if cz else wz0)
                        g = gx + gy + (biz + cz)
                        idx_s[j, pl.ds(k * L, L)] = jnp.where(v, g, fb)
                        w_s[j, pl.ds(k * L, L)] = jnp.where(v, w, 0.0)
                    pltpu.async_copy(
                        w_s.at[j], chunk.at[idx_s.at[j]], sem, add=True)

                    @pl.when(j >= 3)
                    def _wait_lag():
                        pltpu.make_async_copy(
                            w_s.at[j - 3], chunk.at[idx_s.at[j - 3]],
                            sem).wait()
                    return carry3

                lax.fori_loop(0, ng, group_body, 0)

                def drain_body(jt, carry3):
                    pltpu.make_async_copy(
                        w_s.at[jt], chunk.at[idx_s.at[jt]], sem).wait()
                    return carry3
                lax.fori_loop(jnp.maximum(ng - 3, 0), ng, drain_body, 0)
                return carry2

            lax.fori_loop(0, NB, block_body, 0)
            plsc.subcore_barrier()

            def out_body(i, carry2):
                gs = qcs + i

                @pl.when(gs < N1)
                def _copy():
                    tw = SLICE_W // NS
                    pltpu.sync_copy(
                        chunk.at[pl.ds(i * SLICE_W + s * tw, tw)],
                        out.at[pl.ds(gs * SLICE_W + s * tw, tw)])
                return carry2

            lax.fori_loop(0, CS, out_body, 0)
            plsc.subcore_barrier()
            return carry

        lax.fori_loop(0, NP, pass_body, 0)

    return grid_kernel


def kernel(rho, xs):
    m = rho.shape[0]
    nb = -(-m // (NS * B))
    mp = nb * B * NS
    pad = mp - m
    rho_p = jnp.concatenate([rho, jnp.zeros((pad,), jnp.float32)])
    xs_p = jnp.concatenate([xs, jnp.zeros((pad, 3), jnp.float32)])
    xst = xs_p.T
    vol = _make_grid_kernel(mp)(xst[0], xst[1], xst[2], rho_p)
    return vol.reshape(N1, N2, N3)


# speedup vs baseline: 22.8610x; 1.0127x over previous
import functools

import jax
import jax.numpy as jnp
from jax import lax
from jax.experimental import pallas as pl
from jax.experimental.pallas import tpu as pltpu
from jax.experimental.pallas import tpu_sc as plsc

N1 = N2 = N3 = 256
SLICE_W = N2 * N3
NC, NS, L = 2, 16, 16
NW = NC * NS
CS = 26
NQ = -(-N1 // CS)
NP = -(-NQ // NC)
CHUNK = CS * SLICE_W
B = 1024
G = B // L


def _floor16(pos):
    t = pos.astype(jnp.int32)
    tf = t.astype(jnp.float32)
    gt = tf > pos
    bi = jnp.where(gt, t - 1, t)
    bf = jnp.where(gt, tf - 1.0, tf)
    return bi, pos - bf


def _make_grid_kernel(MP):
    NB = MP // (NS * B)
    PPW = NB * B
    mesh = plsc.VectorSubcoreMesh(
        core_axis_name="c", subcore_axis_name="s",
        num_cores=NC, num_subcores=NS)

    @functools.partial(
        pl.kernel,
        out_type=jax.ShapeDtypeStruct((N1 * N2 * N3,), jnp.float32),
        mesh=mesh,
        scratch_types=[
            pltpu.VMEM_SHARED((CHUNK,), jnp.float32),
            pltpu.VMEM((B,), jnp.float32),
            pltpu.VMEM((B,), jnp.float32),
            pltpu.VMEM((B,), jnp.float32),
            pltpu.VMEM((B,), jnp.float32),
            pltpu.VMEM((G, 8 * L), jnp.int32),
            pltpu.VMEM((G, 8 * L), jnp.float32),
            pltpu.VMEM((2048,), jnp.float32),
            pltpu.VMEM((B,), jnp.int32),
            pltpu.SemaphoreType.DMA,
            pltpu.SemaphoreType.DMA,
        ],
        compiler_params=pltpu.CompilerParams(needs_layout_passes=False),
    )
    def grid_kernel(xh, yh, zh, rho_p, out, chunk, xv, yv, zv, rv, idx_s,
                    w_s, zbuf, cidx, sem, lsem):
        c = lax.axis_index("c")
        s = lax.axis_index("s")
        iota = lax.iota(jnp.int32, L)
        zeros16 = jnp.zeros((L,), jnp.float32)

        def zb_body(i, carry):
            zbuf[pl.ds(i * L, L)] = zeros16
            return carry
        lax.fori_loop(0, 2048 // L, zb_body, 0)

        def pass_body(p, carry):
            q = p * NC + c
            cb = q * CHUNK
            qcs = q * CS
            limx = jnp.minimum(CS, N1 - qcs)

            def zc_body(i, carry2):
                pltpu.sync_copy(
                    zbuf,
                    chunk.at[pl.ds(s * (CHUNK // NS) + i * 2048, 2048)])
                return carry2
            lax.fori_loop(0, CHUNK // NS // 2048, zc_body, 0)
            plsc.subcore_barrier()

            fb_base = s * (CHUNK // NS)

            def block_body(b, carry2):
                base = s * PPW + b * B
                bsl = pl.ds(base, B)
                d1 = pltpu.async_copy(xh.at[bsl], xv, lsem)
                d2 = pltpu.async_copy(yh.at[bsl], yv, lsem)
                d3 = pltpu.async_copy(zh.at[bsl], zv, lsem)
                d4 = pltpu.async_copy(rho_p.at[bsl], rv, lsem)
                d1.wait()
                d2.wait()
                d3.wait()
                d4.wait()

                def scan_body(j, cnt_v):
                    x = xv[pl.ds(j * L, L)]
                    pos = x * 256.0 - 0.5
                    t = pos.astype(jnp.int32)
                    tf = t.astype(jnp.float32)
                    bix = jnp.where(tf > pos, t - 1, t)
                    t0 = bix - qcs
                    rel = (t0 >= -1) & (t0 < limx)
                    plsc.store_scatter(
                        cidx, [iota * G + cnt_v], j * L + iota, mask=rel)
                    return cnt_v + rel.astype(jnp.int32)

                cnt_v = lax.fori_loop(0, G, scan_body,
                                      jnp.zeros((L,), jnp.int32))
                ng = jnp.max(cnt_v)

                def group_body(j, carry3):
                    inb = cnt_v > j
                    ids = jnp.where(
                        inb, plsc.load_gather(cidx, [iota * G + j]), 0)
                    x = plsc.load_gather(xv, [ids])
                    y = plsc.load_gather(yv, [ids])
                    z = plsc.load_gather(zv, [ids])
                    r = plsc.load_gather(rv, [ids])
                    bix, dx = _floor16(x * 256.0 - 0.5)
                    biy, dy = _floor16(y * 256.0 - 0.5)
                    biz, dz = _floor16(z * 256.0 - 0.5)
                    wx0 = 1.0 - dx
                    ax0 = r * wx0
                    ax1 = r * dx
                    wy0 = 1.0 - dy
                    wz0 = 1.0 - dz
                    t0 = bix - qcs
                    t1 = t0 + 1
                    vx0 = (t0 >= 0) & (t0 < limx)
                    vx1 = (t1 >= 0) & (t1 < limx)
                    vy0 = (biy >= 0) & (biy < N2)
                    vy1 = (biy >= -1) & (biy < N2 - 1)
                    vz0 = (biz >= 0) & (biz < N3)
                    vz1 = (biz >= -1) & (biz < N3 - 1)
                    gx0 = (bix << 16) - cb
                    gx1 = gx0 + SLICE_W
                    gy0 = biy << 8
                    gy1 = gy0 + N3
                    fb = fb_base + j * L + iota
                    vmask = inb
                    for k, (vx, vy, vz, gx, gy, axw, wy) in enumerate((
                            (vx0, vy0, vz0, gx0, gy0, ax0, wy0),
                            (vx0, vy0, vz1, gx0, gy0, ax0, wy0),
                            (vx0, vy1, vz0, gx0, gy1, ax0, dy),
                            (vx0, vy1, vz1, gx0, gy1, ax0, dy),
                            (vx1, vy0, vz0, gx1, gy0, ax1, wy0),
                            (vx1, vy0, vz1, gx1, gy0, ax1, wy0),
                            (vx1, vy1, vz0, gx1, gy1, ax1, dy),
                            (vx1, vy1, vz1, gx1, gy1, ax1, dy),
                    )):
                        cz = k & 1
                        v = vx & vy & vz & vmask
                        w = (axw * wy) * (dz if cz else wz0)
                        g = gx + gy + (biz + cz)
                        idx_s[j, pl.ds(k * L, L)] = jnp.where(v, g, fb)
                        w_s[j, pl.ds(k * L, L)] = jnp.where(v, w, 0.0)
                    pltpu.async_copy(
                        w_s.at[j], chunk.at[idx_s.at[j]], sem, add=True)

                    @pl.when(j >= 3)
                    def _wait_lag():
                        pltpu.make_async_copy(
                            w_s.at[j - 3], chunk.at[idx_s.at[j - 3]],
                            sem).wait()
                    return carry3

                lax.fori_loop(0, ng, group_body, 0)

                def drain_body(jt, carry3):
                    pltpu.make_async_copy(
                        w_s.at[jt], chunk.at[idx_s.at[jt]], sem).wait()
                    return carry3
                lax.fori_loop(jnp.maximum(ng - 3, 0), ng, drain_body, 0)
                return carry2

            lax.fori_loop(0, NB, block_body, 0)
            plsc.subcore_barrier()

            def out_body(i, carry2):
                gs = qcs + i

                @pl.when(gs < N1)
                def _copy():
                    tw = SLICE_W // NS
                    pltpu.sync_copy(
                        chunk.at[pl.ds(i * SLICE_W + s * tw, tw)],
                        out.at[pl.ds(gs * SLICE_W + s * tw, tw)])
                return carry2

            lax.fori_loop(0, CS, out_body, 0)
            plsc.subcore_barrier()
            return carry

        lax.fori_loop(0, NP, pass_body, 0)

    return grid_kernel


def kernel(rho, xs):
    m = rho.shape[0]
    nb = -(-m // (NS * B))
    mp = nb * B * NS
    pad = mp - m
    rho_p = jnp.concatenate([rho, jnp.zeros((pad,), jnp.float32)])
    xs_p = jnp.concatenate([xs, jnp.zeros((pad, 3), jnp.float32)])
    xst = xs_p.T
    vol = _make_grid_kernel(mp)(xst[0], xst[1], xst[2], rho_p)
    return vol.reshape(N1, N2, N3)
